# Initial kernel scaffold; baseline (speedup 1.0000x reference)
import jax
import jax.numpy as jnp
from jax.experimental import pallas as pl


def kernel(x, som, fc1_w, fc1_b, fc2_w, fc2_b, fc3_w, fc3_b, fc4_w, fc4_b):
    raise NotImplementedError("write your pallas kernel here")


# speedup vs baseline: 2.2112x; 2.2112x over previous
import functools

import jax
import jax.numpy as jnp
from jax.experimental import pallas as pl

B = 128
IMG = 32
H, W = 16, 16
C, KH, KW = 3, 3, 3
OUT_HW = IMG - KH + 1
NPOS = OUT_HW * OUT_HW
K = H * W
D = C * KH * KW
DP = 32
ROWS = B * NPOS
ROW_BLK = 3600


def _som_kernel(p_ref, ct_ref, mn_ref):
    p = p_ref[...]
    ct = ct_ref[...]
    dot = jnp.dot(p, ct, preferred_element_type=jnp.float32)
    s2 = jnp.sum(p * p, axis=1, keepdims=True)
    c2 = jnp.sum(ct * ct, axis=0, keepdims=True)
    v = s2 - 2.0 * dot + c2
    err = jnp.sqrt(jnp.maximum(v, 0.0) * (1.0 / D))
    minv = jnp.min(err, axis=1, keepdims=True)
    iota = jax.lax.broadcasted_iota(jnp.int32, err.shape, 1)
    idx = jnp.min(jnp.where(err == minv, iota, K), axis=1, keepdims=True)
    m = (idx >> 4).astype(jnp.float32) * (1.0 / H)
    n = (idx & 15).astype(jnp.float32) * (1.0 / W)
    mn_ref[:, 0:1] = m
    mn_ref[:, 1:2] = n


def _mlp_kernel(h_ref, w1_ref, b1_ref, w2_ref, b2_ref, w3_ref, b3_ref,
                w4_ref, b4_ref, out_ref):
    h = h_ref[...]
    a = jnp.maximum(jnp.dot(h, w1_ref[...], preferred_element_type=jnp.float32)
                    + b1_ref[...], 0.0)
    a = jnp.maximum(jnp.dot(a, w2_ref[...], preferred_element_type=jnp.float32)
                    + b2_ref[...], 0.0)
    a = jnp.maximum(jnp.dot(a, w3_ref[...], preferred_element_type=jnp.float32)
                    + b3_ref[...], 0.0)
    z = jnp.dot(a, w4_ref[...], preferred_element_type=jnp.float32) + b4_ref[...]
    zmax = jnp.max(z, axis=1, keepdims=True)
    zs = z - zmax
    out_ref[...] = zs - jnp.log(jnp.sum(jnp.exp(zs), axis=1, keepdims=True))


@jax.jit
def kernel(x, som, fc1_w, fc1_b, fc2_w, fc2_b, fc3_w, fc3_b, fc4_w, fc4_b):
    slices = [x[:, :, dj:dj + OUT_HW, dk:dk + OUT_HW]
              for dj in range(KH) for dk in range(KW)]
    st = jnp.stack(slices, axis=2)
    patches = st.transpose(0, 3, 4, 1, 2).reshape(ROWS, D)
    patches = jnp.pad(patches, ((0, 0), (0, DP - D)))
    code_t = jnp.pad(som.reshape(K, D).T, ((0, DP - D), (0, 0)))

    mn = pl.pallas_call(
        _som_kernel,
        grid=(ROWS // ROW_BLK,),
        in_specs=[
            pl.BlockSpec((ROW_BLK, DP), lambda i: (i, 0)),
            pl.BlockSpec((DP, K), lambda i: (0, 0)),
        ],
        out_specs=pl.BlockSpec((ROW_BLK, 2), lambda i: (i, 0)),
        out_shape=jax.ShapeDtypeStruct((ROWS, 2), jnp.float32),
    )(patches, code_t)

    h = mn.reshape(B, NPOS, 2).transpose(0, 2, 1).reshape(B, 2 * NPOS)

    out = pl.pallas_call(
        _mlp_kernel,
        out_shape=jax.ShapeDtypeStruct((B, 10), jnp.float32),
    )(h, fc1_w.T, fc1_b.reshape(1, -1), fc2_w.T, fc2_b.reshape(1, -1),
      fc3_w.T, fc3_b.reshape(1, -1), fc4_w.T, fc4_b.reshape(1, -1))
    return out


# speedup vs baseline: 2.4998x; 1.1305x over previous
import functools

import jax
import jax.numpy as jnp
from jax.experimental import pallas as pl

B = 128
IMG = 32
H, W = 16, 16
C, KH, KW = 3, 3, 3
OUT_HW = IMG - KH + 1
NPOS = OUT_HW * OUT_HW
K = H * W
D = C * KH * KW
DP = 32
ROWS = B * NPOS
ROW_BLK = 3600


def _som_kernel(p_ref, ct_ref, mn_ref):
    p = p_ref[...]
    ct = ct_ref[...]
    dot = jnp.dot(p, ct, preferred_element_type=jnp.float32)
    c2 = jnp.sum(ct * ct, axis=0, keepdims=True)
    v = c2 - 2.0 * dot
    minv = jnp.min(v, axis=1, keepdims=True)
    iota = jax.lax.broadcasted_iota(jnp.int32, v.shape, 1)
    idx = jnp.min(jnp.where(v == minv, iota, K), axis=1, keepdims=True)
    m = (idx >> 4).astype(jnp.float32) * (1.0 / H)
    n = (idx & 15).astype(jnp.float32) * (1.0 / W)
    mn_ref[:, 0:1] = m
    mn_ref[:, 1:2] = n


def _mlp_kernel(h_ref, w1_ref, b1_ref, w2_ref, b2_ref, w3_ref, b3_ref,
                w4_ref, b4_ref, out_ref):
    h = h_ref[...]
    a = jnp.maximum(jnp.dot(h, w1_ref[...], preferred_element_type=jnp.float32)
                    + b1_ref[...], 0.0)
    a = jnp.maximum(jnp.dot(a, w2_ref[...], preferred_element_type=jnp.float32)
                    + b2_ref[...], 0.0)
    a = jnp.maximum(jnp.dot(a, w3_ref[...], preferred_element_type=jnp.float32)
                    + b3_ref[...], 0.0)
    z = jnp.dot(a, w4_ref[...], preferred_element_type=jnp.float32) + b4_ref[...]
    zmax = jnp.max(z, axis=1, keepdims=True)
    zs = z - zmax
    out_ref[...] = zs - jnp.log(jnp.sum(jnp.exp(zs), axis=1, keepdims=True))


@jax.jit
def kernel(x, som, fc1_w, fc1_b, fc2_w, fc2_b, fc3_w, fc3_b, fc4_w, fc4_b):
    slices = [x[:, :, dj:dj + OUT_HW, dk:dk + OUT_HW]
              for dj in range(KH) for dk in range(KW)]
    st = jnp.stack(slices, axis=2)
    patches = st.transpose(0, 3, 4, 1, 2).reshape(ROWS, D)
    patches = jnp.pad(patches, ((0, 0), (0, DP - D)))
    code_t = jnp.pad(som.reshape(K, D).T, ((0, DP - D), (0, 0)))

    mn = pl.pallas_call(
        _som_kernel,
        grid=(ROWS // ROW_BLK,),
        in_specs=[
            pl.BlockSpec((ROW_BLK, DP), lambda i: (i, 0)),
            pl.BlockSpec((DP, K), lambda i: (0, 0)),
        ],
        out_specs=pl.BlockSpec((ROW_BLK, 2), lambda i: (i, 0)),
        out_shape=jax.ShapeDtypeStruct((ROWS, 2), jnp.float32),
    )(patches, code_t)

    h = mn.reshape(B, NPOS, 2).transpose(0, 2, 1).reshape(B, 2 * NPOS)

    out = pl.pallas_call(
        _mlp_kernel,
        out_shape=jax.ShapeDtypeStruct((B, 10), jnp.float32),
    )(h, fc1_w.T, fc1_b.reshape(1, -1), fc2_w.T, fc2_b.reshape(1, -1),
      fc3_w.T, fc3_b.reshape(1, -1), fc4_w.T, fc4_b.reshape(1, -1))
    return out


# speedup vs baseline: 5.8373x; 2.3351x over previous
import numpy as np

import jax
import jax.numpy as jnp
from jax.experimental import pallas as pl

B = 128
IMG = 32
H, W = 16, 16
C, KH, KW = 3, 3, 3
OUT_HW = IMG - KH + 1
NPOS = OUT_HW * OUT_HW
K = H * W
D = C * KH * KW
CHUNK = 8
RB = CHUNK * IMG


def _build_static():
    e = np.zeros((C, IMG, C * IMG), np.float32)
    for c in range(C):
        for kp in range(IMG):
            e[c, kp, C * kp + c] = 1.0
    sh = np.zeros((KH, IMG, IMG), np.float32)
    for dj in range(KH):
        for j in range(IMG - dj):
            sh[dj, j, j + dj] = 1.0
    s = np.stack([np.kron(np.eye(CHUNK, dtype=np.float32), sh[dj])
                  for dj in range(KH)])
    g = np.zeros((KH, C * IMG, IMG * 32), np.float32)
    for dj in range(KH):
        for k in range(OUT_HW):
            for dk in range(KW):
                for c in range(C):
                    g[dj, C * (k + dk) + c, 32 * k + dj * 9 + dk * 3 + c] = 1.0
    return e, s, g


_E, _S, _G = _build_static()
_PAT = np.zeros((1, 1024), np.float32)
_PAT[0, 31::32] = 1.0
_XT4 = np.zeros((4 * K, 4), np.float32)
for _kk in range(4):
    _XT4[_kk * K:(_kk + 1) * K, _kk] = np.arange(K, dtype=np.float32)


def _som_kernel(x_ref, e_ref, s_ref, g_ref, bd_ref, pat_ref, xt4_ref,
                m_ref, n_ref):
    xb = x_ref[...]
    xt = None
    for c in range(C):
        xc = xb[:, c].reshape(RB, IMG)
        t = jnp.dot(xc, e_ref[c], preferred_element_type=jnp.float32)
        xt = t if xt is None else xt + t
    p = None
    for dj in range(KH):
        td = jnp.dot(s_ref[dj], xt, preferred_element_type=jnp.float32)
        q = jnp.dot(td, g_ref[dj], preferred_element_type=jnp.float32)
        p = q if p is None else p + q
    p = p + pat_ref[...]
    bd = bd_ref[...]
    xt4 = xt4_ref[...]
    idx_cols = []
    for g in range(8):
        dg = jnp.dot(p[:, g * 128:(g + 1) * 128], bd,
                     preferred_element_type=jnp.float32)
        msks = []
        for kk in range(4):
            sl = dg[:, kk * K:(kk + 1) * K]
            minv = jnp.min(sl, axis=1, keepdims=True)
            msks.append((sl == minv).astype(jnp.float32))
        idx_cols.append(jnp.dot(jnp.concatenate(msks, axis=1), xt4,
                                preferred_element_type=jnp.float32))
    idx = jnp.concatenate(idx_cols, axis=1)
    mf = jnp.floor(idx * (1.0 / W))
    m_ref[...] = mf * (1.0 / H)
    n_ref[...] = (idx - W * mf) * (1.0 / W)


def _dot_nt(a, w):
    return jax.lax.dot_general(a, w, (((1,), (1,)), ((), ())),
                               preferred_element_type=jnp.float32)


def _mlp_kernel(m_ref, n_ref, w1m_ref, w1n_ref, b1_ref, w2_ref, b2_ref,
                w3_ref, b3_ref, w4_ref, b4_ref, out_ref):
    a = (_dot_nt(m_ref[...], w1m_ref[...]) + _dot_nt(n_ref[...], w1n_ref[...])
         + b1_ref[...])
    a = jnp.maximum(a, 0.0)
    a = jnp.maximum(_dot_nt(a, w2_ref[...]) + b2_ref[...], 0.0)
    a = jnp.maximum(_dot_nt(a, w3_ref[...]) + b3_ref[...], 0.0)
    z = _dot_nt(a, w4_ref[...]) + b4_ref[...]
    zmax = jnp.max(z, axis=1, keepdims=True)
    zs = z - zmax
    out_ref[...] = zs - jnp.log(jnp.sum(jnp.exp(zs), axis=1, keepdims=True))


@jax.jit
def kernel(x, som, fc1_w, fc1_b, fc2_w, fc2_b, fc3_w, fc3_b, fc4_w, fc4_b):
    code = som.reshape(K, C, KH, KW).transpose(0, 2, 3, 1).reshape(K, D)
    code_p = jnp.pad(code, ((0, 0), (0, 32 - D)))
    c2 = jnp.sum(code_p * code_p, axis=1)
    bd = jnp.zeros((4, 32, 4, K), jnp.float32)
    for kk in range(4):
        bd = bd.at[kk, :, kk, :].set(-2.0 * code_p.T)
        bd = bd.at[kk, 31, kk, :].set(c2)
    bd = bd.reshape(128, 4 * K)
    pat = jnp.asarray(_PAT)
    xt4 = jnp.asarray(_XT4)

    m, n = pl.pallas_call(
        _som_kernel,
        grid=(B // CHUNK,),
        in_specs=[
            pl.BlockSpec((CHUNK, C, IMG, IMG), lambda i: (i, 0, 0, 0)),
            pl.BlockSpec(_E.shape, lambda i: (0, 0, 0)),
            pl.BlockSpec(_S.shape, lambda i: (0, 0, 0)),
            pl.BlockSpec(_G.shape, lambda i: (0, 0, 0)),
            pl.BlockSpec((128, 1024), lambda i: (0, 0)),
            pl.BlockSpec((1, 1024), lambda i: (0, 0)),
            pl.BlockSpec((4 * K, 4), lambda i: (0, 0)),
        ],
        out_specs=[pl.BlockSpec((RB, 32), lambda i: (i, 0)),
                   pl.BlockSpec((RB, 32), lambda i: (i, 0))],
        out_shape=[jax.ShapeDtypeStruct((B * IMG, 32), jnp.float32),
                   jax.ShapeDtypeStruct((B * IMG, 32), jnp.float32)],
    )(x, jnp.asarray(_E), jnp.asarray(_S), jnp.asarray(_G), bd, pat, xt4)

    m_r = m.reshape(B, IMG * 32)
    n_r = n.reshape(B, IMG * 32)
    w1 = jnp.pad(fc1_w.reshape(1000, 2, OUT_HW, OUT_HW),
                 ((0, 0), (0, 0), (0, 2), (0, 2)))
    w1m = w1[:, 0].reshape(1000, IMG * 32)
    w1n = w1[:, 1].reshape(1000, IMG * 32)

    out = pl.pallas_call(
        _mlp_kernel,
        out_shape=jax.ShapeDtypeStruct((B, 10), jnp.float32),
    )(m_r, n_r, w1m, w1n, fc1_b.reshape(1, -1), fc2_w, fc2_b.reshape(1, -1),
      fc3_w, fc3_b.reshape(1, -1), fc4_w, fc4_b.reshape(1, -1))
    return out


# speedup vs baseline: 6.4239x; 1.1005x over previous
import numpy as np

import jax
import jax.numpy as jnp
from jax.experimental import pallas as pl
from jax.experimental.pallas import tpu as pltpu

B = 128
IMG = 32
H, W = 16, 16
C, KH, KW = 3, 3, 3
OUT_HW = IMG - KH + 1
NPOS = OUT_HW * OUT_HW
K = H * W
D = C * KH * KW
CHUNK = 8
RB = CHUNK * IMG


def _build_static():
    e = np.zeros((C, IMG, C * IMG), np.float32)
    for c in range(C):
        for kp in range(IMG):
            e[c, kp, C * kp + c] = 1.0
    sh = np.zeros((KH, IMG, IMG), np.float32)
    for dj in range(KH):
        for j in range(IMG - dj):
            sh[dj, j, j + dj] = 1.0
    s = np.stack([np.kron(np.eye(CHUNK, dtype=np.float32), sh[dj])
                  for dj in range(KH)])
    g = np.zeros((KH, C * IMG, IMG * 32), np.float32)
    for dj in range(KH):
        for k in range(OUT_HW):
            for dk in range(KW):
                for c in range(C):
                    g[dj, C * (k + dk) + c, 32 * k + dj * 9 + dk * 3 + c] = 1.0
    return e, s, g


_E, _S, _G = _build_static()
_PAT = np.zeros((1, 1024), np.float32)
_PAT[0, 31::32] = 1.0
_XT4 = np.zeros((4 * K, 4), np.float32)
for _kk in range(4):
    _XT4[_kk * K:(_kk + 1) * K, _kk] = np.arange(K, dtype=np.float32)
_PSEL = np.zeros((IMG * 32, NPOS), np.float32)
for _j in range(OUT_HW):
    for _k in range(OUT_HW):
        _PSEL[_j * 32 + _k, _j * OUT_HW + _k] = 1.0


def _som_kernel(x_ref, e_ref, g_ref, bd_ref, pat_ref, xt4_ref,
                m_ref, n_ref):
    xb = x_ref[...]
    xt = None
    for c in range(C):
        xc = xb[:, c].reshape(RB, IMG)
        t = jnp.dot(xc, e_ref[c], preferred_element_type=jnp.float32)
        xt = t if xt is None else xt + t
    p = None
    for dj in range(KH):
        td = xt if dj == 0 else pltpu.roll(xt, RB - dj, 0)
        q = jnp.dot(td, g_ref[dj], preferred_element_type=jnp.float32)
        p = q if p is None else p + q
    p = p + pat_ref[...]
    bd = bd_ref[...]
    xt4 = xt4_ref[...]
    idx_cols = []
    for g in range(8):
        dg = jnp.dot(p[:, g * 128:(g + 1) * 128], bd,
                     preferred_element_type=jnp.float32)
        msks = []
        for kk in range(4):
            sl = dg[:, kk * K:(kk + 1) * K]
            minv = jnp.min(sl, axis=1, keepdims=True)
            msks.append((sl == minv).astype(jnp.float32))
        idx_cols.append(jnp.dot(jnp.concatenate(msks, axis=1), xt4,
                                preferred_element_type=jnp.float32))
    idx = jnp.concatenate(idx_cols, axis=1)
    mf = jnp.floor(idx * (1.0 / W))
    m_ref[...] = mf * (1.0 / H)
    n_ref[...] = (idx - W * mf) * (1.0 / W)


def _dot_nt(a, w):
    return jax.lax.dot_general(a, w, (((1,), (1,)), ((), ())),
                               preferred_element_type=jnp.float32)


def _mlp_kernel(m_ref, n_ref, psel_ref, w1a_ref, w1b_ref, b1_ref, w2_ref,
                b2_ref, w3_ref, b3_ref, w4_ref, b4_ref, out_ref):
    psel = psel_ref[...]
    hm = jnp.dot(m_ref[...].astype(jnp.bfloat16), psel,
                 preferred_element_type=jnp.float32)
    hn = jnp.dot(n_ref[...].astype(jnp.bfloat16), psel,
                 preferred_element_type=jnp.float32)
    a = _dot_nt(hm, w1a_ref[...]) + _dot_nt(hn, w1b_ref[...]) + b1_ref[...]
    a = jnp.maximum(a, 0.0)
    a = jnp.maximum(_dot_nt(a, w2_ref[...]) + b2_ref[...], 0.0)
    a = jnp.maximum(_dot_nt(a, w3_ref[...]) + b3_ref[...], 0.0)
    z = _dot_nt(a, w4_ref[...]) + b4_ref[...]
    zmax = jnp.max(z, axis=1, keepdims=True)
    zs = z - zmax
    out_ref[...] = zs - jnp.log(jnp.sum(jnp.exp(zs), axis=1, keepdims=True))


@jax.jit
def kernel(x, som, fc1_w, fc1_b, fc2_w, fc2_b, fc3_w, fc3_b, fc4_w, fc4_b):
    code = som.reshape(K, C, KH, KW).transpose(0, 2, 3, 1).reshape(K, D)
    code_p = jnp.pad(code, ((0, 0), (0, 32 - D)))
    c2 = jnp.sum(code_p * code_p, axis=1)
    cpt = jnp.concatenate([-2.0 * code_p.T[:31], c2[None, :]], axis=0)
    bd = jnp.kron(jnp.eye(4, dtype=jnp.float32), cpt)
    pat = jnp.asarray(_PAT)
    xt4 = jnp.asarray(_XT4)

    m, n = pl.pallas_call(
        _som_kernel,
        grid=(B // CHUNK,),
        in_specs=[
            pl.BlockSpec((CHUNK, C, IMG, IMG), lambda i: (i, 0, 0, 0)),
            pl.BlockSpec(_E.shape, lambda i: (0, 0, 0)),
            pl.BlockSpec(_G.shape, lambda i: (0, 0, 0)),
            pl.BlockSpec((128, 1024), lambda i: (0, 0)),
            pl.BlockSpec((1, 1024), lambda i: (0, 0)),
            pl.BlockSpec((4 * K, 4), lambda i: (0, 0)),
        ],
        out_specs=[pl.BlockSpec((RB, 32), lambda i: (i, 0)),
                   pl.BlockSpec((RB, 32), lambda i: (i, 0))],
        out_shape=[jax.ShapeDtypeStruct((B * IMG, 32), jnp.float32),
                   jax.ShapeDtypeStruct((B * IMG, 32), jnp.float32)],
    )(x, jnp.asarray(_E), jnp.asarray(_G), bd, pat, xt4)

    m_r = m.reshape(B, IMG * 32)
    n_r = n.reshape(B, IMG * 32)
    out = pl.pallas_call(
        _mlp_kernel,
        out_shape=jax.ShapeDtypeStruct((B, 10), jnp.float32),
    )(m_r, n_r, jnp.asarray(_PSEL).astype(jnp.bfloat16),
      fc1_w[:, :NPOS], fc1_w[:, NPOS:], fc1_b.reshape(1, -1),
      fc2_w, fc2_b.reshape(1, -1), fc3_w, fc3_b.reshape(1, -1),
      fc4_w, fc4_b.reshape(1, -1))
    return out


# speedup vs baseline: 6.8754x; 1.0703x over previous
import numpy as np

import jax
import jax.numpy as jnp
from jax.experimental import pallas as pl
from jax.experimental.pallas import tpu as pltpu

B = 128
IMG = 32
H, W = 16, 16
C, KH, KW = 3, 3, 3
OUT_HW = IMG - KH + 1
NPOS = OUT_HW * OUT_HW
K = H * W
D = C * KH * KW
CHUNK = 8
RB = CHUNK * IMG
NSTEP = B // CHUNK


def _build_static():
    e = np.zeros((C, IMG, C * IMG), np.float32)
    for c in range(C):
        for kp in range(IMG):
            e[c, kp, C * kp + c] = 1.0
    g = np.zeros((KH, C * IMG, IMG * 32), np.float32)
    for dj in range(KH):
        for k in range(OUT_HW):
            for dk in range(KW):
                for c in range(C):
                    g[dj, C * (k + dk) + c, 32 * k + dj * 9 + dk * 3 + c] = 1.0
    return e, g


_E, _G = _build_static()
_PAT = np.zeros((1, 1024), np.float32)
_PAT[0, 31::32] = 1.0
_XT4 = np.zeros((4 * K, 4), np.float32)
for _kk in range(4):
    _XT4[_kk * K:(_kk + 1) * K, _kk] = np.arange(K, dtype=np.float32)
_PSEL = np.zeros((IMG * 32, NPOS), np.float32)
for _j in range(OUT_HW):
    for _k in range(OUT_HW):
        _PSEL[_j * 32 + _k, _j * OUT_HW + _k] = 1.0


def _dot_nt(a, w):
    return jax.lax.dot_general(a, w, (((1,), (1,)), ((), ())),
                               preferred_element_type=jnp.float32)


def _fused_kernel(x_ref, e_ref, g_ref, bd_ref, pat_ref, xt4_ref, psel_ref,
                  w1a_ref, w1b_ref, b1_ref, w2_ref, b2_ref, w3_ref, b3_ref,
                  w4_ref, b4_ref, out_ref, msc_ref, nsc_ref):
    i = pl.program_id(0)
    xb = x_ref[...]
    xt = None
    for c in range(C):
        xc = xb[:, c].reshape(RB, IMG)
        t = jnp.dot(xc, e_ref[c], preferred_element_type=jnp.float32)
        xt = t if xt is None else xt + t
    p = None
    for dj in range(KH):
        td = xt if dj == 0 else pltpu.roll(xt, RB - dj, 0)
        q = jnp.dot(td, g_ref[dj], preferred_element_type=jnp.float32)
        p = q if p is None else p + q
    p = p + pat_ref[...]
    bd = bd_ref[...]
    xt4 = xt4_ref[...]
    idx_cols = []
    for g in range(8):
        dg = jnp.dot(p[:, g * 128:(g + 1) * 128], bd,
                     preferred_element_type=jnp.float32)
        msks = []
        for kk in range(4):
            sl = dg[:, kk * K:(kk + 1) * K]
            minv = jnp.min(sl, axis=1, keepdims=True)
            msks.append((sl == minv).astype(jnp.float32))
        idx_cols.append(jnp.dot(jnp.concatenate(msks, axis=1), xt4,
                                preferred_element_type=jnp.float32))
    idx = jnp.concatenate(idx_cols, axis=1)
    mf = jnp.floor(idx * (1.0 / W))
    msc_ref[pl.ds(i * RB, RB), :] = mf * (1.0 / H)
    nsc_ref[pl.ds(i * RB, RB), :] = (idx - W * mf) * (1.0 / W)

    @pl.when(i == NSTEP - 1)
    def _mlp():
        psel = psel_ref[...]
        m3 = msc_ref[...].reshape(B, IMG, 32)
        n3 = nsc_ref[...].reshape(B, IMG, 32)
        hmg = jnp.concatenate([m3[:, j, :] for j in range(IMG)], axis=1)
        hng = jnp.concatenate([n3[:, j, :] for j in range(IMG)], axis=1)
        hm = jnp.dot(hmg.astype(jnp.bfloat16), psel,
                     preferred_element_type=jnp.float32)
        hn = jnp.dot(hng.astype(jnp.bfloat16), psel,
                     preferred_element_type=jnp.float32)
        a = _dot_nt(hm, w1a_ref[...]) + _dot_nt(hn, w1b_ref[...]) + b1_ref[...]
        a = jnp.maximum(a, 0.0)
        a = jnp.maximum(_dot_nt(a, w2_ref[...]) + b2_ref[...], 0.0)
        a = jnp.maximum(_dot_nt(a, w3_ref[...]) + b3_ref[...], 0.0)
        z = _dot_nt(a, w4_ref[...]) + b4_ref[...]
        zmax = jnp.max(z, axis=1, keepdims=True)
        zs = z - zmax
        out_ref[...] = zs - jnp.log(jnp.sum(jnp.exp(zs), axis=1,
                                            keepdims=True))


@jax.jit
def kernel(x, som, fc1_w, fc1_b, fc2_w, fc2_b, fc3_w, fc3_b, fc4_w, fc4_b):
    code = som.reshape(K, C, KH, KW).transpose(0, 2, 3, 1).reshape(K, D)
    code_p = jnp.pad(code, ((0, 0), (0, 32 - D)))
    c2 = jnp.sum(code_p * code_p, axis=1)
    cpt = jnp.concatenate([-2.0 * code_p.T[:31], c2[None, :]], axis=0)
    bd = jnp.kron(jnp.eye(4, dtype=jnp.float32), cpt)

    out = pl.pallas_call(
        _fused_kernel,
        grid=(NSTEP,),
        in_specs=[
            pl.BlockSpec((CHUNK, C, IMG, IMG), lambda i: (i, 0, 0, 0)),
            pl.BlockSpec(_E.shape, lambda i: (0, 0, 0)),
            pl.BlockSpec(_G.shape, lambda i: (0, 0, 0)),
            pl.BlockSpec((128, 1024), lambda i: (0, 0)),
            pl.BlockSpec((1, 1024), lambda i: (0, 0)),
            pl.BlockSpec((4 * K, 4), lambda i: (0, 0)),
            pl.BlockSpec((IMG * 32, NPOS), lambda i: (0, 0)),
            pl.BlockSpec((1000, NPOS), lambda i: (0, 0)),
            pl.BlockSpec((1000, NPOS), lambda i: (0, 0)),
            pl.BlockSpec((1, 1000), lambda i: (0, 0)),
            pl.BlockSpec((500, 1000), lambda i: (0, 0)),
            pl.BlockSpec((1, 500), lambda i: (0, 0)),
            pl.BlockSpec((200, 500), lambda i: (0, 0)),
            pl.BlockSpec((1, 200), lambda i: (0, 0)),
            pl.BlockSpec((10, 200), lambda i: (0, 0)),
            pl.BlockSpec((1, 10), lambda i: (0, 0)),
        ],
        out_specs=pl.BlockSpec((B, 10), lambda i: (0, 0)),
        out_shape=jax.ShapeDtypeStruct((B, 10), jnp.float32),
        scratch_shapes=[pltpu.VMEM((B * IMG, 32), jnp.float32),
                        pltpu.VMEM((B * IMG, 32), jnp.float32)],
    )(x, jnp.asarray(_E), jnp.asarray(_G), bd, jnp.asarray(_PAT),
      jnp.asarray(_XT4), jnp.asarray(_PSEL).astype(jnp.bfloat16),
      fc1_w[:, :NPOS], fc1_w[:, NPOS:], fc1_b.reshape(1, -1),
      fc2_w, fc2_b.reshape(1, -1), fc3_w, fc3_b.reshape(1, -1),
      fc4_w, fc4_b.reshape(1, -1))
    return out


# speedup vs baseline: 7.7884x; 1.1328x over previous
import numpy as np

import jax
import jax.numpy as jnp
from jax.experimental import pallas as pl
from jax.experimental.pallas import tpu as pltpu

B = 128
IMG = 32
H, W = 16, 16
C, KH, KW = 3, 3, 3
OUT_HW = IMG - KH + 1
NPOS = OUT_HW * OUT_HW
K = H * W
D = C * KH * KW
CHUNK = 8
RB = CHUNK * IMG
NSTEP = B // CHUNK


def _build_static():
    e = np.zeros((C, IMG, C * IMG), np.float32)
    for c in range(C):
        for kp in range(IMG):
            e[c, kp, C * kp + c] = 1.0
    g = np.zeros((KH, C * IMG, IMG * 32), np.float32)
    for dj in range(KH):
        for k in range(OUT_HW):
            for dk in range(KW):
                for c in range(C):
                    g[dj, C * (k + dk) + c, 32 * k + dj * 9 + dk * 3 + c] = 1.0
    return e, g


_E, _G = _build_static()
_PAT = np.zeros((1, 1024), np.float32)
_PAT[0, 31::32] = 1.0
_XT4 = np.zeros((4 * K, 4), np.float32)
for _kk in range(4):
    _XT4[_kk * K:(_kk + 1) * K, _kk] = np.arange(K, dtype=np.float32)
_PSEL = np.zeros((IMG * 32, NPOS), np.float32)
for _j in range(OUT_HW):
    for _k in range(OUT_HW):
        _PSEL[_j * 32 + _k, _j * OUT_HW + _k] = 1.0


def _dot_nt(a, w):
    return jax.lax.dot_general(a, w, (((1,), (1,)), ((), ())),
                               preferred_element_type=jnp.float32)


def _fused_kernel(x_ref, e_ref, g_ref, bd_ref, pat_ref, xt4_ref, psel_ref,
                  w1a_ref, w1b_ref, b1_ref, w2_ref, b2_ref, w3_ref, b3_ref,
                  w4_ref, b4_ref, out_ref, msc_ref, nsc_ref):
    i = pl.program_id(0)
    xb = x_ref[...]
    xt = None
    for c in range(C):
        xc = xb[:, c].reshape(RB, IMG)
        t = jnp.dot(xc, e_ref[c], preferred_element_type=jnp.float32)
        xt = t if xt is None else xt + t
    p = None
    for dj in range(KH):
        td = xt if dj == 0 else pltpu.roll(xt, RB - dj, 0)
        q = jnp.dot(td, g_ref[dj], preferred_element_type=jnp.float32)
        p = q if p is None else p + q
    p = p + pat_ref[...]
    bd = bd_ref[...]
    xt4 = xt4_ref[...]
    iotaf = jax.lax.broadcasted_iota(jnp.int32, (RB, K), 1).astype(jnp.float32)
    idx_cols = []
    for g in range(8):
        nkk = 2 if g == 7 else 4
        dg = jnp.dot(p[:, g * 128:(g + 1) * 128], bd[:, :nkk * K],
                     preferred_element_type=jnp.float32)
        if g % 2 == 0:
            msks = []
            for kk in range(nkk):
                sl = dg[:, kk * K:(kk + 1) * K]
                minv = jnp.min(sl, axis=1, keepdims=True)
                msks.append((sl == minv).astype(jnp.float32))
            idx_cols.append(jnp.dot(jnp.concatenate(msks, axis=1),
                                    xt4[:nkk * K, :nkk],
                                    preferred_element_type=jnp.float32))
        else:
            cols = []
            for kk in range(nkk):
                sl = dg[:, kk * K:(kk + 1) * K]
                minv = jnp.min(sl, axis=1, keepdims=True)
                cols.append(jnp.sum(jnp.where(sl == minv, iotaf, 0.0),
                                    axis=1, keepdims=True))
            idx_cols.append(jnp.concatenate(cols, axis=1))
    idx_cols.append(jnp.zeros((RB, 2), jnp.float32))
    idx = jnp.concatenate(idx_cols, axis=1)
    mf = jnp.floor(idx * (1.0 / W))
    msc_ref[pl.ds(i * RB, RB), :] = mf * (1.0 / H)
    nsc_ref[pl.ds(i * RB, RB), :] = (idx - W * mf) * (1.0 / W)

    @pl.when(i == NSTEP - 1)
    def _mlp():
        psel = psel_ref[...]
        m3 = msc_ref[...].reshape(B, IMG, 32)
        n3 = nsc_ref[...].reshape(B, IMG, 32)
        hmg = jnp.concatenate([m3[:, j, :] for j in range(IMG)], axis=1)
        hng = jnp.concatenate([n3[:, j, :] for j in range(IMG)], axis=1)
        hm = jnp.dot(hmg.astype(jnp.bfloat16), psel,
                     preferred_element_type=jnp.float32)
        hn = jnp.dot(hng.astype(jnp.bfloat16), psel,
                     preferred_element_type=jnp.float32)
        a = _dot_nt(hm, w1a_ref[...]) + _dot_nt(hn, w1b_ref[...]) + b1_ref[...]
        a = jnp.maximum(a, 0.0)
        a = jnp.maximum(_dot_nt(a, w2_ref[...]) + b2_ref[...], 0.0)
        a = jnp.maximum(_dot_nt(a, w3_ref[...]) + b3_ref[...], 0.0)
        z = _dot_nt(a, w4_ref[...]) + b4_ref[...]
        zmax = jnp.max(z, axis=1, keepdims=True)
        zs = z - zmax
        out_ref[...] = zs - jnp.log(jnp.sum(jnp.exp(zs), axis=1,
                                            keepdims=True))


@jax.jit
def kernel(x, som, fc1_w, fc1_b, fc2_w, fc2_b, fc3_w, fc3_b, fc4_w, fc4_b):
    code = som.reshape(K, C, KH, KW).transpose(0, 2, 3, 1).reshape(K, D)
    code_p = jnp.pad(code, ((0, 0), (0, 32 - D)))
    c2 = jnp.sum(code_p * code_p, axis=1)
    cpt = jnp.concatenate([-2.0 * code_p.T[:31], c2[None, :]], axis=0)
    bd = jnp.kron(jnp.eye(4, dtype=jnp.float32), cpt)

    out = pl.pallas_call(
        _fused_kernel,
        grid=(NSTEP,),
        in_specs=[
            pl.BlockSpec((CHUNK, C, IMG, IMG), lambda i: (i, 0, 0, 0)),
            pl.BlockSpec(_E.shape, lambda i: (0, 0, 0)),
            pl.BlockSpec(_G.shape, lambda i: (0, 0, 0)),
            pl.BlockSpec((128, 1024), lambda i: (0, 0)),
            pl.BlockSpec((1, 1024), lambda i: (0, 0)),
            pl.BlockSpec((4 * K, 4), lambda i: (0, 0)),
            pl.BlockSpec((IMG * 32, NPOS), lambda i: (0, 0)),
            pl.BlockSpec((1000, NPOS), lambda i: (0, 0)),
            pl.BlockSpec((1000, NPOS), lambda i: (0, 0)),
            pl.BlockSpec((1, 1000), lambda i: (0, 0)),
            pl.BlockSpec((500, 1000), lambda i: (0, 0)),
            pl.BlockSpec((1, 500), lambda i: (0, 0)),
            pl.BlockSpec((200, 500), lambda i: (0, 0)),
            pl.BlockSpec((1, 200), lambda i: (0, 0)),
            pl.BlockSpec((10, 200), lambda i: (0, 0)),
            pl.BlockSpec((1, 10), lambda i: (0, 0)),
        ],
        out_specs=pl.BlockSpec((B, 10), lambda i: (0, 0)),
        out_shape=jax.ShapeDtypeStruct((B, 10), jnp.float32),
        scratch_shapes=[pltpu.VMEM((B * IMG, 32), jnp.float32),
                        pltpu.VMEM((B * IMG, 32), jnp.float32)],
    )(x, jnp.asarray(_E), jnp.asarray(_G), bd, jnp.asarray(_PAT),
      jnp.asarray(_XT4), jnp.asarray(_PSEL).astype(jnp.bfloat16),
      fc1_w[:, :NPOS], fc1_w[:, NPOS:], fc1_b.reshape(1, -1),
      fc2_w, fc2_b.reshape(1, -1), fc3_w, fc3_b.reshape(1, -1),
      fc4_w, fc4_b.reshape(1, -1))
    return out
